# Initial kernel scaffold; baseline (speedup 1.0000x reference)
import functools

import jax
import jax.numpy as jnp
from jax import lax
from jax.experimental import pallas as pl
from jax.experimental.pallas import tpu as pltpu

N_HASHES = 8
BUCKET_SIZE = 64
N_BUCKETS = 64
S = 4096
D = 1024
HASH_TILE = 256
N_CHUNKS = N_HASHES * N_BUCKETS


def _hash_kernel(qk_ref, rot_ref, buckets_ref, keys_ref):
    s_blk = pl.program_id(1)
    x = qk_ref[0]
    r = rot_ref[0]
    rr = lax.dot_general(x, r, (((1,), (0,)), ((), ())),
                         preferred_element_type=jnp.float32,
                         precision=lax.Precision.HIGHEST)
    lane = lax.broadcasted_iota(jnp.int32, (HASH_TILE, N_BUCKETS // 2), 1)
    t_vec = s_blk * HASH_TILE + lax.broadcasted_iota(jnp.int32, (HASH_TILE,), 0)
    half = N_BUCKETS // 2
    for h in range(N_HASHES):
        sub = rr[:, h * half:(h + 1) * half]
        mx = jnp.max(sub, axis=1, keepdims=True)
        mn = jnp.min(sub, axis=1, keepdims=True)
        pos = jnp.min(jnp.where(sub == mx, lane, N_BUCKETS), axis=1)
        neg = jnp.min(jnp.where(sub == mn, lane, N_BUCKETS), axis=1)
        b_loc = jnp.where(mx[:, 0] >= -mn[:, 0], pos, half + neg)
        buckets_ref[0, h, :] = b_loc + h * N_BUCKETS
        keys_ref[0, h, :] = b_loc * S + t_vec


def _hash_stage(qk, rot):
    B = qk.shape[0]
    grid = (B, S // HASH_TILE)
    buckets, keys = pl.pallas_call(
        _hash_kernel,
        grid=grid,
        in_specs=[
            pl.BlockSpec((1, HASH_TILE, D), lambda b, s: (b, s, 0)),
            pl.BlockSpec((1, D, N_HASHES * (N_BUCKETS // 2)), lambda b, s: (b, 0, 0)),
        ],
        out_specs=[
            pl.BlockSpec((1, N_HASHES, HASH_TILE), lambda b, s: (b, 0, s)),
            pl.BlockSpec((1, N_HASHES, HASH_TILE), lambda b, s: (b, 0, s)),
        ],
        out_shape=[
            jax.ShapeDtypeStruct((B, N_HASHES, S), jnp.int32),
            jax.ShapeDtypeStruct((B, N_HASHES, S), jnp.int32),
        ],
    )(qk, rot)
    return buckets, keys


def _attn_kernel(sqk_c, sqk_p, sv_c, sv_p, st_c, st_p, so_ref, slog_ref):
    q = sqk_c[0, 0]
    k = jnp.concatenate([sqk_c[0, 0], sqk_p[0, 0]], axis=0)
    nrm = jax.lax.rsqrt(jnp.sum(k * k, axis=1, keepdims=True) + 1e-6)
    kn = k * nrm
    v = jnp.concatenate([sv_c[0, 0], sv_p[0, 0]], axis=0)
    dots = lax.dot_general(q, kn, (((1,), (1,)), ((), ())),
                           preferred_element_type=jnp.float32,
                           precision=lax.Precision.HIGHEST) * (D ** -0.5)
    qt = st_c[0, 0, 0]
    kt = jnp.concatenate([st_c[0, 0, 0], st_p[0, 0, 0]])
    dots = jnp.where(qt[:, None] == kt[None, :], -50000.0, dots)
    mx = jnp.max(dots, axis=1, keepdims=True)
    e = jnp.exp(dots - mx)
    ssum = jnp.sum(e, axis=1, keepdims=True)
    p = e / ssum
    so_ref[0, 0] = lax.dot_general(p, v, (((1,), (0,)), ((), ())),
                                   preferred_element_type=jnp.float32,
                                   precision=lax.Precision.HIGHEST)
    slog_ref[0, 0, 0] = (jnp.log(ssum) + mx)[:, 0]


def _attn_stage(sqk, sv, st):
    B = sqk.shape[0]
    grid = (B, N_CHUNKS)
    data_spec_c = pl.BlockSpec((1, 1, BUCKET_SIZE, D), lambda b, i: (b, i, 0, 0))
    data_spec_p = pl.BlockSpec((1, 1, BUCKET_SIZE, D),
                               lambda b, i: (b, (i - 1) % N_CHUNKS, 0, 0))
    st_spec_c = pl.BlockSpec((1, 1, 1, BUCKET_SIZE), lambda b, i: (b, i, 0, 0))
    st_spec_p = pl.BlockSpec((1, 1, 1, BUCKET_SIZE),
                             lambda b, i: (b, (i - 1) % N_CHUNKS, 0, 0))
    so, slog = pl.pallas_call(
        _attn_kernel,
        grid=grid,
        in_specs=[data_spec_c, data_spec_p, data_spec_c, data_spec_p,
                  st_spec_c, st_spec_p],
        out_specs=[
            pl.BlockSpec((1, 1, BUCKET_SIZE, D), lambda b, i: (b, i, 0, 0)),
            pl.BlockSpec((1, 1, 1, BUCKET_SIZE), lambda b, i: (b, i, 0, 0)),
        ],
        out_shape=[
            jax.ShapeDtypeStruct((B, N_CHUNKS, BUCKET_SIZE, D), jnp.float32),
            jax.ShapeDtypeStruct((B, N_CHUNKS, 1, BUCKET_SIZE), jnp.float32),
        ],
    )(sqk, sqk, sv, sv, st, st)
    return so, slog


def kernel(qk, v, seed_):
    B = qk.shape[0]
    rot = jax.random.normal(jax.random.key(seed_),
                            (B, D, N_HASHES, N_BUCKETS // 2), dtype=qk.dtype)
    rot2 = rot.reshape(B, D, N_HASHES * (N_BUCKETS // 2))

    buckets, keys = _hash_stage(qk, rot2)
    buckets_out = buckets.reshape(B, N_HASHES * S)

    st_local = jnp.argsort(keys, axis=-1).astype(jnp.int32)
    inv = jnp.argsort(st_local, axis=-1).astype(jnp.int32)

    st_flat = st_local.reshape(B, N_HASHES * S)
    sqk = jnp.take_along_axis(qk, st_flat[:, :, None], axis=1)
    sv = jnp.take_along_axis(v, st_flat[:, :, None], axis=1)

    sqk = sqk.reshape(B, N_CHUNKS, BUCKET_SIZE, D)
    sv = sv.reshape(B, N_CHUNKS, BUCKET_SIZE, D)
    st4 = st_flat.reshape(B, N_CHUNKS, 1, BUCKET_SIZE)

    so, slog = _attn_stage(sqk, sv, st4)
    so_flat = so.reshape(B, N_HASHES * S, D)
    slog_flat = slog.reshape(B, N_HASHES * S)

    undo = (inv + (jnp.arange(N_HASHES, dtype=jnp.int32) * S)[None, :, None]
            ).reshape(B, N_HASHES * S)
    o = jnp.take_along_axis(so_flat, undo[:, :, None], axis=1)
    logits = jnp.take_along_axis(slog_flat, undo, axis=1)
    o = o.reshape(B, N_HASHES, S, D)
    logits = logits.reshape(B, N_HASHES, S, 1)
    lse = jax.scipy.special.logsumexp(logits, axis=1, keepdims=True)
    probs = jnp.exp(logits - lse)
    out = jnp.sum(o * probs, axis=1)
    return out, buckets_out


# speedup vs baseline: 1215.7666x; 1215.7666x over previous
import functools

import jax
import jax.numpy as jnp
from jax import lax
from jax.experimental import pallas as pl
from jax.experimental.pallas import tpu as pltpu

N_HASHES = 8
BUCKET_SIZE = 64
N_BUCKETS = 64
S = 4096
D = 1024
HASH_TILE = 256
N_CHUNKS = N_HASHES * N_BUCKETS


def _hash_kernel(qk_ref, rot_ref, buckets_ref, keys_ref):
    s_blk = pl.program_id(1)
    x = qk_ref[0]
    r = rot_ref[0]
    rr = lax.dot_general(x, r, (((1,), (0,)), ((), ())),
                         preferred_element_type=jnp.float32,
                         precision=lax.Precision.DEFAULT)
    lane = lax.broadcasted_iota(jnp.int32, (HASH_TILE, N_BUCKETS // 2), 1)
    t_vec = s_blk * HASH_TILE + lax.broadcasted_iota(jnp.int32, (HASH_TILE,), 0)
    half = N_BUCKETS // 2
    for h in range(N_HASHES):
        sub = rr[:, h * half:(h + 1) * half]
        mx = jnp.max(sub, axis=1, keepdims=True)
        mn = jnp.min(sub, axis=1, keepdims=True)
        pos = jnp.min(jnp.where(sub == mx, lane, N_BUCKETS), axis=1)
        neg = jnp.min(jnp.where(sub == mn, lane, N_BUCKETS), axis=1)
        b_loc = jnp.where(mx[:, 0] >= -mn[:, 0], pos, half + neg)
        buckets_ref[0, h, :] = b_loc + h * N_BUCKETS
        keys_ref[0, h, :] = b_loc * S + t_vec


def _hash_stage(qk, rot):
    B = qk.shape[0]
    grid = (B, S // HASH_TILE)
    buckets, keys = pl.pallas_call(
        _hash_kernel,
        grid=grid,
        in_specs=[
            pl.BlockSpec((1, HASH_TILE, D), lambda b, s: (b, s, 0)),
            pl.BlockSpec((1, D, N_HASHES * (N_BUCKETS // 2)), lambda b, s: (b, 0, 0)),
        ],
        out_specs=[
            pl.BlockSpec((1, N_HASHES, HASH_TILE), lambda b, s: (b, 0, s)),
            pl.BlockSpec((1, N_HASHES, HASH_TILE), lambda b, s: (b, 0, s)),
        ],
        out_shape=[
            jax.ShapeDtypeStruct((B, N_HASHES, S), jnp.int32),
            jax.ShapeDtypeStruct((B, N_HASHES, S), jnp.int32),
        ],
    )(qk, rot)
    return buckets, keys


def _attn_kernel(sqk_c, sqk_p, sv_c, sv_p, st_c, st_p, so_ref, slog_ref):
    q = sqk_c[0, 0]
    k = jnp.concatenate([sqk_c[0, 0], sqk_p[0, 0]], axis=0)
    nrm = jax.lax.rsqrt(jnp.sum(k * k, axis=1, keepdims=True) + 1e-6)
    kn = k * nrm
    v = jnp.concatenate([sv_c[0, 0], sv_p[0, 0]], axis=0)
    dots = lax.dot_general(q, kn, (((1,), (1,)), ((), ())),
                           preferred_element_type=jnp.float32,
                           precision=lax.Precision.HIGHEST) * (D ** -0.5)
    qt = st_c[0, 0, 0]
    kt = jnp.concatenate([st_c[0, 0, 0], st_p[0, 0, 0]])
    dots = jnp.where(qt[:, None] == kt[None, :], -50000.0, dots)
    mx = jnp.max(dots, axis=1, keepdims=True)
    e = jnp.exp(dots - mx)
    ssum = jnp.sum(e, axis=1, keepdims=True)
    p = e / ssum
    so_ref[0, 0] = lax.dot_general(p, v, (((1,), (0,)), ((), ())),
                                   preferred_element_type=jnp.float32,
                                   precision=lax.Precision.HIGHEST)
    slog_ref[0, 0, 0] = (jnp.log(ssum) + mx)[:, 0]


def _attn_stage(sqk, sv, st):
    B = sqk.shape[0]
    grid = (B, N_CHUNKS)
    data_spec_c = pl.BlockSpec((1, 1, BUCKET_SIZE, D), lambda b, i: (b, i, 0, 0))
    data_spec_p = pl.BlockSpec((1, 1, BUCKET_SIZE, D),
                               lambda b, i: (b, (i - 1) % N_CHUNKS, 0, 0))
    st_spec_c = pl.BlockSpec((1, 1, 1, BUCKET_SIZE), lambda b, i: (b, i, 0, 0))
    st_spec_p = pl.BlockSpec((1, 1, 1, BUCKET_SIZE),
                             lambda b, i: (b, (i - 1) % N_CHUNKS, 0, 0))
    so, slog = pl.pallas_call(
        _attn_kernel,
        grid=grid,
        in_specs=[data_spec_c, data_spec_p, data_spec_c, data_spec_p,
                  st_spec_c, st_spec_p],
        out_specs=[
            pl.BlockSpec((1, 1, BUCKET_SIZE, D), lambda b, i: (b, i, 0, 0)),
            pl.BlockSpec((1, 1, 1, BUCKET_SIZE), lambda b, i: (b, i, 0, 0)),
        ],
        out_shape=[
            jax.ShapeDtypeStruct((B, N_CHUNKS, BUCKET_SIZE, D), jnp.float32),
            jax.ShapeDtypeStruct((B, N_CHUNKS, 1, BUCKET_SIZE), jnp.float32),
        ],
    )(sqk, sqk, sv, sv, st, st)
    return so, slog


def kernel(qk, v, seed_):
    B = qk.shape[0]
    rot = jax.random.normal(jax.random.key(seed_),
                            (B, D, N_HASHES, N_BUCKETS // 2), dtype=qk.dtype)
    rot2 = rot.reshape(B, D, N_HASHES * (N_BUCKETS // 2))

    buckets, keys = _hash_stage(qk, rot2)
    buckets_out = buckets.reshape(B, N_HASHES * S)

    st_local = jnp.argsort(keys, axis=-1).astype(jnp.int32)
    inv = jnp.argsort(st_local, axis=-1).astype(jnp.int32)

    st_flat = st_local.reshape(B, N_HASHES * S)
    sqk = jnp.take_along_axis(qk, st_flat[:, :, None], axis=1)
    sv = jnp.take_along_axis(v, st_flat[:, :, None], axis=1)

    sqk = sqk.reshape(B, N_CHUNKS, BUCKET_SIZE, D)
    sv = sv.reshape(B, N_CHUNKS, BUCKET_SIZE, D)
    st4 = st_flat.reshape(B, N_CHUNKS, 1, BUCKET_SIZE)

    so, slog = _attn_stage(sqk, sv, st4)
    so_flat = so.reshape(B, N_HASHES * S, D)
    slog_flat = slog.reshape(B, N_HASHES * S)

    undo = (inv + (jnp.arange(N_HASHES, dtype=jnp.int32) * S)[None, :, None]
            ).reshape(B, N_HASHES * S)
    o = jnp.take_along_axis(so_flat, undo[:, :, None], axis=1)
    logits = jnp.take_along_axis(slog_flat, undo, axis=1)
    o = o.reshape(B, N_HASHES, S, D)
    logits = logits.reshape(B, N_HASHES, S, 1)
    lse = jax.scipy.special.logsumexp(logits, axis=1, keepdims=True)
    probs = jnp.exp(logits - lse)
    out = jnp.sum(o * probs, axis=1)
    return out, buckets_out


# speedup vs baseline: 2509.6518x; 2.0643x over previous
import functools

import jax
import jax.numpy as jnp
from jax import lax
from jax.experimental import pallas as pl
from jax.experimental.pallas import tpu as pltpu
from jax.experimental.pallas import tpu_sc as plsc

N_HASHES = 8
BUCKET_SIZE = 64
N_BUCKETS = 64
S = 4096
D = 1024
HASH_TILE = 256
N_CHUNKS = N_HASHES * N_BUCKETS


def _hash_kernel(qk_ref, rot_ref, buckets_ref, keys_ref):
    s_blk = pl.program_id(1)
    x = qk_ref[0]
    r = rot_ref[0]
    rr = lax.dot_general(x, r, (((1,), (0,)), ((), ())),
                         preferred_element_type=jnp.float32,
                         precision=lax.Precision.DEFAULT)
    lane = lax.broadcasted_iota(jnp.int32, (HASH_TILE, N_BUCKETS // 2), 1)
    t_vec = s_blk * HASH_TILE + lax.broadcasted_iota(jnp.int32, (HASH_TILE,), 0)
    half = N_BUCKETS // 2
    for h in range(N_HASHES):
        sub = rr[:, h * half:(h + 1) * half]
        mx = jnp.max(sub, axis=1, keepdims=True)
        mn = jnp.min(sub, axis=1, keepdims=True)
        pos = jnp.min(jnp.where(sub == mx, lane, N_BUCKETS), axis=1)
        neg = jnp.min(jnp.where(sub == mn, lane, N_BUCKETS), axis=1)
        b_loc = jnp.where(mx[:, 0] >= -mn[:, 0], pos, half + neg)
        buckets_ref[0, h, :] = b_loc + h * N_BUCKETS
        keys_ref[0, h, :] = b_loc * S + t_vec


def _hash_stage(qk, rot):
    B = qk.shape[0]
    grid = (B, S // HASH_TILE)
    buckets, keys = pl.pallas_call(
        _hash_kernel,
        grid=grid,
        in_specs=[
            pl.BlockSpec((1, HASH_TILE, D), lambda b, s: (b, s, 0)),
            pl.BlockSpec((1, D, N_HASHES * (N_BUCKETS // 2)), lambda b, s: (b, 0, 0)),
        ],
        out_specs=[
            pl.BlockSpec((1, N_HASHES, HASH_TILE), lambda b, s: (b, 0, s)),
            pl.BlockSpec((1, N_HASHES, HASH_TILE), lambda b, s: (b, 0, s)),
        ],
        out_shape=[
            jax.ShapeDtypeStruct((B, N_HASHES, S), jnp.int32),
            jax.ShapeDtypeStruct((B, N_HASHES, S), jnp.int32),
        ],
    )(qk, rot)
    return buckets, keys


_N_WORKERS = 32
_GC = 32


def _sc_gather_body(qk_hbm, v_hbm, idx_hbm, sqk_hbm, sv_hbm,
                    idx_v, qkr, vr, sem_q, sem_v):
    n_rows = sqk_hbm.shape[0]
    rows_per_w = n_rows // _N_WORKERS
    wid = lax.axis_index("s") * 2 + lax.axis_index("c")
    base = wid * rows_per_w

    def step(c, carry):
        off = base + c * _GC
        pltpu.sync_copy(idx_hbm.at[pl.ds(off, _GC)], idx_v)
        cq = pltpu.async_copy(qk_hbm.at[idx_v], qkr, sem_q)
        cv = pltpu.async_copy(v_hbm.at[idx_v], vr, sem_v)
        cq.wait()
        cv.wait()
        pltpu.sync_copy(qkr, sqk_hbm.at[pl.ds(off, _GC)])
        pltpu.sync_copy(vr, sv_hbm.at[pl.ds(off, _GC)])
        return carry

    lax.fori_loop(0, rows_per_w // _GC, step, 0)


def _sc_gather(qk2, v2, idx):
    n = idx.shape[0]
    f = jax.ShapeDtypeStruct((n, D), jnp.float32)
    run = pl.kernel(
        _sc_gather_body,
        out_type=[f, f],
        mesh=plsc.VectorSubcoreMesh(core_axis_name="c", subcore_axis_name="s"),
        scratch_types=[
            pltpu.VMEM((_GC,), jnp.int32),
            pltpu.VMEM((_GC, D), jnp.float32),
            pltpu.VMEM((_GC, D), jnp.float32),
            pltpu.SemaphoreType.DMA,
            pltpu.SemaphoreType.DMA,
        ],
    )
    return run(qk2, v2, idx)


def _attn_kernel(sqk_c, sqk_p, sv_c, sv_p, st_c, st_p, so_ref, slog_ref):
    q = sqk_c[0, 0]
    k = jnp.concatenate([sqk_c[0, 0], sqk_p[0, 0]], axis=0)
    nrm = jax.lax.rsqrt(jnp.sum(k * k, axis=1, keepdims=True) + 1e-6)
    kn = k * nrm
    v = jnp.concatenate([sv_c[0, 0], sv_p[0, 0]], axis=0)
    dots = lax.dot_general(q, kn, (((1,), (1,)), ((), ())),
                           preferred_element_type=jnp.float32,
                           precision=lax.Precision.HIGHEST) * (D ** -0.5)
    qt = st_c[0, 0, 0]
    kt = jnp.concatenate([st_c[0, 0, 0], st_p[0, 0, 0]])
    dots = jnp.where(qt[:, None] == kt[None, :], -50000.0, dots)
    mx = jnp.max(dots, axis=1, keepdims=True)
    e = jnp.exp(dots - mx)
    ssum = jnp.sum(e, axis=1, keepdims=True)
    p = e / ssum
    so_ref[0, 0] = lax.dot_general(p, v, (((1,), (0,)), ((), ())),
                                   preferred_element_type=jnp.float32,
                                   precision=lax.Precision.HIGHEST)
    slog_ref[0, 0, 0] = (jnp.log(ssum) + mx)[:, 0]


def _attn_stage(sqk, sv, st):
    B = sqk.shape[0]
    grid = (B, N_CHUNKS)
    data_spec_c = pl.BlockSpec((1, 1, BUCKET_SIZE, D), lambda b, i: (b, i, 0, 0))
    data_spec_p = pl.BlockSpec((1, 1, BUCKET_SIZE, D),
                               lambda b, i: (b, (i - 1) % N_CHUNKS, 0, 0))
    st_spec_c = pl.BlockSpec((1, 1, 1, BUCKET_SIZE), lambda b, i: (b, i, 0, 0))
    st_spec_p = pl.BlockSpec((1, 1, 1, BUCKET_SIZE),
                             lambda b, i: (b, (i - 1) % N_CHUNKS, 0, 0))
    so, slog = pl.pallas_call(
        _attn_kernel,
        grid=grid,
        in_specs=[data_spec_c, data_spec_p, data_spec_c, data_spec_p,
                  st_spec_c, st_spec_p],
        out_specs=[
            pl.BlockSpec((1, 1, BUCKET_SIZE, D), lambda b, i: (b, i, 0, 0)),
            pl.BlockSpec((1, 1, 1, BUCKET_SIZE), lambda b, i: (b, i, 0, 0)),
        ],
        out_shape=[
            jax.ShapeDtypeStruct((B, N_CHUNKS, BUCKET_SIZE, D), jnp.float32),
            jax.ShapeDtypeStruct((B, N_CHUNKS, 1, BUCKET_SIZE), jnp.float32),
        ],
    )(sqk, sqk, sv, sv, st, st)
    return so, slog


def kernel(qk, v, seed_):
    B = qk.shape[0]
    rot = jax.random.normal(jax.random.key(seed_),
                            (B, D, N_HASHES, N_BUCKETS // 2), dtype=qk.dtype)
    rot2 = rot.reshape(B, D, N_HASHES * (N_BUCKETS // 2))

    buckets, keys = _hash_stage(qk, rot2)
    buckets_out = buckets.reshape(B, N_HASHES * S)

    st_local = jnp.argsort(keys, axis=-1).astype(jnp.int32)
    inv = jnp.argsort(st_local, axis=-1).astype(jnp.int32)

    st_flat = st_local.reshape(B, N_HASHES * S)
    gidx = (st_local
            + (jnp.arange(B, dtype=jnp.int32) * S)[:, None, None]).reshape(-1)
    sqk_f, sv_f = _sc_gather(qk.reshape(B * S, D), v.reshape(B * S, D), gidx)

    sqk = sqk_f.reshape(B, N_CHUNKS, BUCKET_SIZE, D)
    sv = sv_f.reshape(B, N_CHUNKS, BUCKET_SIZE, D)
    st4 = st_flat.reshape(B, N_CHUNKS, 1, BUCKET_SIZE)

    so, slog = _attn_stage(sqk, sv, st4)
    so_flat = so.reshape(B, N_HASHES * S, D)
    slog_flat = slog.reshape(B, N_HASHES * S)

    undo = (inv + (jnp.arange(N_HASHES, dtype=jnp.int32) * S)[None, :, None]
            ).reshape(B, N_HASHES * S)
    o = jnp.take_along_axis(so_flat, undo[:, :, None], axis=1)
    logits = jnp.take_along_axis(slog_flat, undo, axis=1)
    o = o.reshape(B, N_HASHES, S, D)
    logits = logits.reshape(B, N_HASHES, S, 1)
    lse = jax.scipy.special.logsumexp(logits, axis=1, keepdims=True)
    probs = jnp.exp(logits - lse)
    out = jnp.sum(o * probs, axis=1)
    return out, buckets_out


# speedup vs baseline: 2562.1025x; 1.0209x over previous
import functools

import jax
import jax.numpy as jnp
from jax import lax
from jax.experimental import pallas as pl
from jax.experimental.pallas import tpu as pltpu
from jax.experimental.pallas import tpu_sc as plsc

N_HASHES = 8
BUCKET_SIZE = 64
N_BUCKETS = 64
S = 4096
D = 1024
HASH_TILE = 256
N_CHUNKS = N_HASHES * N_BUCKETS


def _hash_kernel(qk_ref, rot_ref, buckets_ref, keys_ref):
    s_blk = pl.program_id(1)
    x = qk_ref[0]
    r = rot_ref[0]
    rr = lax.dot_general(x, r, (((1,), (0,)), ((), ())),
                         preferred_element_type=jnp.float32,
                         precision=lax.Precision.DEFAULT)
    lane = lax.broadcasted_iota(jnp.int32, (HASH_TILE, N_BUCKETS // 2), 1)
    t_vec = s_blk * HASH_TILE + lax.broadcasted_iota(jnp.int32, (HASH_TILE,), 0)
    half = N_BUCKETS // 2
    for h in range(N_HASHES):
        sub = rr[:, h * half:(h + 1) * half]
        mx = jnp.max(sub, axis=1, keepdims=True)
        mn = jnp.min(sub, axis=1, keepdims=True)
        pos = jnp.min(jnp.where(sub == mx, lane, N_BUCKETS), axis=1)
        neg = jnp.min(jnp.where(sub == mn, lane, N_BUCKETS), axis=1)
        b_loc = jnp.where(mx[:, 0] >= -mn[:, 0], pos, half + neg)
        buckets_ref[0, h, :] = b_loc + h * N_BUCKETS
        keys_ref[0, h, :] = b_loc * S + t_vec


def _hash_stage(qk, rot):
    B = qk.shape[0]
    grid = (B, S // HASH_TILE)
    buckets, keys = pl.pallas_call(
        _hash_kernel,
        grid=grid,
        in_specs=[
            pl.BlockSpec((1, HASH_TILE, D), lambda b, s: (b, s, 0)),
            pl.BlockSpec((1, D, N_HASHES * (N_BUCKETS // 2)), lambda b, s: (b, 0, 0)),
        ],
        out_specs=[
            pl.BlockSpec((1, N_HASHES, HASH_TILE), lambda b, s: (b, 0, s)),
            pl.BlockSpec((1, N_HASHES, HASH_TILE), lambda b, s: (b, 0, s)),
        ],
        out_shape=[
            jax.ShapeDtypeStruct((B, N_HASHES, S), jnp.int32),
            jax.ShapeDtypeStruct((B, N_HASHES, S), jnp.int32),
        ],
    )(qk, rot)
    return buckets, keys


_N_WORKERS = 32
_GC = 32


def _sc_gather_body(qk_hbm, v_hbm, idx_hbm, sqk_hbm, sv_hbm,
                    idx_v, qkr, vr, sem_q, sem_v):
    n_rows = sqk_hbm.shape[0]
    rows_per_w = n_rows // _N_WORKERS
    wid = lax.axis_index("s") * 2 + lax.axis_index("c")
    base = wid * rows_per_w

    def step(c, carry):
        off = base + c * _GC
        pltpu.sync_copy(idx_hbm.at[pl.ds(off, _GC)], idx_v)
        cq = pltpu.async_copy(qk_hbm.at[idx_v], qkr, sem_q)
        cv = pltpu.async_copy(v_hbm.at[idx_v], vr, sem_v)
        cq.wait()
        cv.wait()
        pltpu.sync_copy(qkr, sqk_hbm.at[pl.ds(off, _GC)])
        pltpu.sync_copy(vr, sv_hbm.at[pl.ds(off, _GC)])
        return carry

    lax.fori_loop(0, rows_per_w // _GC, step, 0)


def _sc_gather(qk2, v2, idx):
    n = idx.shape[0]
    f = jax.ShapeDtypeStruct((n, D), jnp.float32)
    run = pl.kernel(
        _sc_gather_body,
        out_type=[f, f],
        mesh=plsc.VectorSubcoreMesh(core_axis_name="c", subcore_axis_name="s"),
        scratch_types=[
            pltpu.VMEM((_GC,), jnp.int32),
            pltpu.VMEM((_GC, D), jnp.float32),
            pltpu.VMEM((_GC, D), jnp.float32),
            pltpu.SemaphoreType.DMA,
            pltpu.SemaphoreType.DMA,
        ],
    )
    return run(qk2, v2, idx)


_TC8 = 8


def _sc_combine_body(idx_hbm, so_hbm, slogw_hbm, out_hbm,
                     idxbuf, rows, lrows, outbuf, sem, seml):
    total_t = out_hbm.shape[0]
    t_per_w = total_t // _N_WORKERS
    wid = lax.axis_index("s") * 2 + lax.axis_index("c")
    tbase = wid * t_per_w

    def chunk_step(c, carry):
        pltpu.sync_copy(idx_hbm.at[pl.ds((tbase + c * _TC8) * N_HASHES,
                                         _TC8 * N_HASHES)], idxbuf)
        cr = pltpu.async_copy(so_hbm.at[idxbuf], rows, sem)
        cl = pltpu.async_copy(slogw_hbm.at[idxbuf], lrows, seml)
        cr.wait()
        cl.wait()
        for tt in range(_TC8):
            ls = [lrows[tt * N_HASHES + h, pl.ds(0, 16)] for h in range(N_HASHES)]
            m = ls[0]
            for h in range(1, N_HASHES):
                m = jnp.maximum(m, ls[h])
            es = [jnp.exp(l - m) for l in ls]
            ssum = es[0]
            for h in range(1, N_HASHES):
                ssum = ssum + es[h]
            ws = [e / ssum for e in es]

            def d_step(dc, carry2):
                sl = pl.ds(dc * 16, 16)
                acc = ws[0] * rows[tt * N_HASHES, sl]
                for h in range(1, N_HASHES):
                    acc = acc + ws[h] * rows[tt * N_HASHES + h, sl]
                outbuf[tt, sl] = acc
                return carry2

            lax.fori_loop(0, D // 16, d_step, 0)
        pltpu.sync_copy(outbuf, out_hbm.at[pl.ds(tbase + c * _TC8, _TC8)])
        return carry

    lax.fori_loop(0, t_per_w // _TC8, chunk_step, 0)


def _sc_combine(idx_o, so_flat, slogw):
    total_t = so_flat.shape[0] // N_HASHES
    run = pl.kernel(
        _sc_combine_body,
        out_type=jax.ShapeDtypeStruct((total_t, D), jnp.float32),
        mesh=plsc.VectorSubcoreMesh(core_axis_name="c", subcore_axis_name="s"),
        scratch_types=[
            pltpu.VMEM((_TC8 * N_HASHES,), jnp.int32),
            pltpu.VMEM((_TC8 * N_HASHES, D), jnp.float32),
            pltpu.VMEM((_TC8 * N_HASHES, 128), jnp.float32),
            pltpu.VMEM((_TC8, D), jnp.float32),
            pltpu.SemaphoreType.DMA,
            pltpu.SemaphoreType.DMA,
        ],
    )
    return run(idx_o, so_flat, slogw)


def _attn_kernel(sqk_c, sqk_p, sv_c, sv_p, st_c, st_p, so_ref, slog_ref):
    q = sqk_c[0, 0]
    k = jnp.concatenate([sqk_c[0, 0], sqk_p[0, 0]], axis=0)
    nrm = jax.lax.rsqrt(jnp.sum(k * k, axis=1, keepdims=True) + 1e-6)
    kn = k * nrm
    v = jnp.concatenate([sv_c[0, 0], sv_p[0, 0]], axis=0)
    dots = lax.dot_general(q, kn, (((1,), (1,)), ((), ())),
                           preferred_element_type=jnp.float32,
                           precision=lax.Precision.HIGHEST) * (D ** -0.5)
    qt = st_c[0, 0, 0]
    kt = jnp.concatenate([st_c[0, 0, 0], st_p[0, 0, 0]])
    dots = jnp.where(qt[:, None] == kt[None, :], -50000.0, dots)
    mx = jnp.max(dots, axis=1, keepdims=True)
    e = jnp.exp(dots - mx)
    ssum = jnp.sum(e, axis=1, keepdims=True)
    p = e / ssum
    so_ref[0, 0] = lax.dot_general(p, v, (((1,), (0,)), ((), ())),
                                   preferred_element_type=jnp.float32,
                                   precision=lax.Precision.HIGHEST)
    slog_ref[0, 0] = jnp.broadcast_to(jnp.log(ssum) + mx, (BUCKET_SIZE, 128))


def _attn_stage(sqk, sv, st):
    B = sqk.shape[0]
    grid = (B, N_CHUNKS)
    data_spec_c = pl.BlockSpec((1, 1, BUCKET_SIZE, D), lambda b, i: (b, i, 0, 0))
    data_spec_p = pl.BlockSpec((1, 1, BUCKET_SIZE, D),
                               lambda b, i: (b, (i - 1) % N_CHUNKS, 0, 0))
    st_spec_c = pl.BlockSpec((1, 1, 1, BUCKET_SIZE), lambda b, i: (b, i, 0, 0))
    st_spec_p = pl.BlockSpec((1, 1, 1, BUCKET_SIZE),
                             lambda b, i: (b, (i - 1) % N_CHUNKS, 0, 0))
    so, slog = pl.pallas_call(
        _attn_kernel,
        grid=grid,
        in_specs=[data_spec_c, data_spec_p, data_spec_c, data_spec_p,
                  st_spec_c, st_spec_p],
        out_specs=[
            pl.BlockSpec((1, 1, BUCKET_SIZE, D), lambda b, i: (b, i, 0, 0)),
            pl.BlockSpec((1, 1, BUCKET_SIZE, 128), lambda b, i: (b, i, 0, 0)),
        ],
        out_shape=[
            jax.ShapeDtypeStruct((B, N_CHUNKS, BUCKET_SIZE, D), jnp.float32),
            jax.ShapeDtypeStruct((B, N_CHUNKS, BUCKET_SIZE, 128), jnp.float32),
        ],
    )(sqk, sqk, sv, sv, st, st)
    return so, slog


def kernel(qk, v, seed_):
    B = qk.shape[0]
    rot = jax.random.normal(jax.random.key(seed_),
                            (B, D, N_HASHES, N_BUCKETS // 2), dtype=qk.dtype)
    rot2 = rot.reshape(B, D, N_HASHES * (N_BUCKETS // 2))

    buckets, keys = _hash_stage(qk, rot2)
    buckets_out = buckets.reshape(B, N_HASHES * S)

    st_local = jnp.argsort(keys, axis=-1).astype(jnp.int32)
    inv = jnp.argsort(st_local, axis=-1).astype(jnp.int32)

    st_flat = st_local.reshape(B, N_HASHES * S)
    gidx = (st_local
            + (jnp.arange(B, dtype=jnp.int32) * S)[:, None, None]).reshape(-1)
    sqk_f, sv_f = _sc_gather(qk.reshape(B * S, D), v.reshape(B * S, D), gidx)

    sqk = sqk_f.reshape(B, N_CHUNKS, BUCKET_SIZE, D)
    sv = sv_f.reshape(B, N_CHUNKS, BUCKET_SIZE, D)
    st4 = st_flat.reshape(B, N_CHUNKS, 1, BUCKET_SIZE)

    so, slog = _attn_stage(sqk, sv, st4)
    so_flat = so.reshape(B * N_HASHES * S, D)
    slogw = slog.reshape(B * N_HASHES * S, 128)

    idx_o = (inv
             + (jnp.arange(N_HASHES, dtype=jnp.int32) * S)[None, :, None]
             + (jnp.arange(B, dtype=jnp.int32) * (N_HASHES * S))[:, None, None])
    idx_o = jnp.transpose(idx_o, (0, 2, 1)).reshape(-1)
    out = _sc_combine(idx_o, so_flat, slogw).reshape(B, S, D)
    return out, buckets_out


# speedup vs baseline: 3355.2697x; 1.3096x over previous
import functools

import jax
import jax.numpy as jnp
from jax import lax
from jax.experimental import pallas as pl
from jax.experimental.pallas import tpu as pltpu
from jax.experimental.pallas import tpu_sc as plsc

N_HASHES = 8
BUCKET_SIZE = 64
N_BUCKETS = 64
S = 4096
D = 1024
HASH_TILE = 256
N_CHUNKS = N_HASHES * N_BUCKETS


def _hash_kernel(qk_ref, rot_ref, buckets_ref, keys_ref):
    s_blk = pl.program_id(1)
    x = qk_ref[0]
    r = rot_ref[0]
    rr = lax.dot_general(x, r, (((1,), (0,)), ((), ())),
                         preferred_element_type=jnp.float32,
                         precision=lax.Precision.DEFAULT)
    lane = lax.broadcasted_iota(jnp.int32, (HASH_TILE, N_BUCKETS // 2), 1)
    t_vec = s_blk * HASH_TILE + lax.broadcasted_iota(jnp.int32, (HASH_TILE,), 0)
    half = N_BUCKETS // 2
    for h in range(N_HASHES):
        sub = rr[:, h * half:(h + 1) * half]
        mx = jnp.max(sub, axis=1, keepdims=True)
        mn = jnp.min(sub, axis=1, keepdims=True)
        pos = jnp.min(jnp.where(sub == mx, lane, N_BUCKETS), axis=1)
        neg = jnp.min(jnp.where(sub == mn, lane, N_BUCKETS), axis=1)
        b_loc = jnp.where(mx[:, 0] >= -mn[:, 0], pos, half + neg)
        buckets_ref[0, h, :] = b_loc + h * N_BUCKETS
        keys_ref[0, h, :] = b_loc * S + t_vec


def _hash_stage(qk, rot):
    B = qk.shape[0]
    grid = (B, S // HASH_TILE)
    buckets, keys = pl.pallas_call(
        _hash_kernel,
        grid=grid,
        in_specs=[
            pl.BlockSpec((1, HASH_TILE, D), lambda b, s: (b, s, 0)),
            pl.BlockSpec((1, D, N_HASHES * (N_BUCKETS // 2)), lambda b, s: (b, 0, 0)),
        ],
        out_specs=[
            pl.BlockSpec((1, N_HASHES, HASH_TILE), lambda b, s: (b, 0, s)),
            pl.BlockSpec((1, N_HASHES, HASH_TILE), lambda b, s: (b, 0, s)),
        ],
        out_shape=[
            jax.ShapeDtypeStruct((B, N_HASHES, S), jnp.int32),
            jax.ShapeDtypeStruct((B, N_HASHES, S), jnp.int32),
        ],
    )(qk, rot)
    return buckets, keys


_N_WORKERS = 32
_GC = 32


def _sc_gather_body(qk_hbm, v_hbm, idx_hbm, sqk_hbm, sv_hbm,
                    idx_v, qkr, vr, sem_q, sem_v):
    n_rows = sqk_hbm.shape[0]
    rows_per_w = n_rows // _N_WORKERS
    wid = lax.axis_index("s") * 2 + lax.axis_index("c")
    base = wid * rows_per_w

    def step(c, carry):
        off = base + c * _GC
        pltpu.sync_copy(idx_hbm.at[pl.ds(off, _GC)], idx_v)
        cq = pltpu.async_copy(qk_hbm.at[idx_v], qkr, sem_q)
        cv = pltpu.async_copy(v_hbm.at[idx_v], vr, sem_v)
        cq.wait()
        cv.wait()
        pltpu.sync_copy(qkr, sqk_hbm.at[pl.ds(off, _GC)])
        pltpu.sync_copy(vr, sv_hbm.at[pl.ds(off, _GC)])
        return carry

    lax.fori_loop(0, rows_per_w // _GC, step, 0)


def _sc_gather(qk2, v2, idx):
    n = idx.shape[0]
    f = jax.ShapeDtypeStruct((n, D), jnp.float32)
    run = pl.kernel(
        _sc_gather_body,
        out_type=[f, f],
        mesh=plsc.VectorSubcoreMesh(core_axis_name="c", subcore_axis_name="s"),
        scratch_types=[
            pltpu.VMEM((_GC,), jnp.int32),
            pltpu.VMEM((_GC, D), jnp.float32),
            pltpu.VMEM((_GC, D), jnp.float32),
            pltpu.SemaphoreType.DMA,
            pltpu.SemaphoreType.DMA,
        ],
    )
    return run(qk2, v2, idx)


_TC8 = 8


def _sc_combine_body(idx_hbm, so_hbm, slogw_hbm, out_hbm,
                     idxbuf, rows, lrows, outbuf, sem, seml):
    total_t = out_hbm.shape[0]
    t_per_w = total_t // _N_WORKERS
    wid = lax.axis_index("s") * 2 + lax.axis_index("c")
    tbase = wid * t_per_w

    def chunk_step(c, carry):
        pltpu.sync_copy(idx_hbm.at[pl.ds((tbase + c * _TC8) * N_HASHES,
                                         _TC8 * N_HASHES)], idxbuf)
        cr = pltpu.async_copy(so_hbm.at[idxbuf], rows, sem)
        cl = pltpu.async_copy(slogw_hbm.at[idxbuf], lrows, seml)
        cr.wait()
        cl.wait()
        for tt in range(_TC8):
            ls = [lrows[tt * N_HASHES + h, pl.ds(0, 16)] for h in range(N_HASHES)]
            m = ls[0]
            for h in range(1, N_HASHES):
                m = jnp.maximum(m, ls[h])
            es = [jnp.exp(l - m) for l in ls]
            ssum = es[0]
            for h in range(1, N_HASHES):
                ssum = ssum + es[h]
            ws = [e / ssum for e in es]

            def d_step(dc, carry2):
                sl = pl.ds(dc * 16, 16)
                acc = ws[0] * rows[tt * N_HASHES, sl]
                for h in range(1, N_HASHES):
                    acc = acc + ws[h] * rows[tt * N_HASHES + h, sl]
                outbuf[tt, sl] = acc
                return carry2

            lax.fori_loop(0, D // 16, d_step, 0)
        pltpu.sync_copy(outbuf, out_hbm.at[pl.ds(tbase + c * _TC8, _TC8)])
        return carry

    lax.fori_loop(0, t_per_w // _TC8, chunk_step, 0)


def _sc_combine(idx_o, so_flat, slogw):
    total_t = so_flat.shape[0] // N_HASHES
    run = pl.kernel(
        _sc_combine_body,
        out_type=jax.ShapeDtypeStruct((total_t, D), jnp.float32),
        mesh=plsc.VectorSubcoreMesh(core_axis_name="c", subcore_axis_name="s"),
        scratch_types=[
            pltpu.VMEM((_TC8 * N_HASHES,), jnp.int32),
            pltpu.VMEM((_TC8 * N_HASHES, D), jnp.float32),
            pltpu.VMEM((_TC8 * N_HASHES, 128), jnp.float32),
            pltpu.VMEM((_TC8, D), jnp.float32),
            pltpu.SemaphoreType.DMA,
            pltpu.SemaphoreType.DMA,
        ],
    )
    return run(idx_o, so_flat, slogw)


def _attn_kernel(sqk_c, sqk_p, sv_c, sv_p, st_c, st_p, so_ref, slog_ref):
    q = sqk_c[0, 0]
    k = jnp.concatenate([sqk_c[0, 0], sqk_p[0, 0]], axis=0)
    nrm = jax.lax.rsqrt(jnp.sum(k * k, axis=1, keepdims=True) + 1e-6)
    kn = k * nrm
    v = jnp.concatenate([sv_c[0, 0], sv_p[0, 0]], axis=0)
    dots = lax.dot_general(q, kn, (((1,), (1,)), ((), ())),
                           preferred_element_type=jnp.float32,
                           precision=lax.Precision.DEFAULT) * (D ** -0.5)
    qt = st_c[0, 0, 0]
    kt = jnp.concatenate([st_c[0, 0, 0], st_p[0, 0, 0]])
    dots = jnp.where(qt[:, None] == kt[None, :], -50000.0, dots)
    mx = jnp.max(dots, axis=1, keepdims=True)
    e = jnp.exp(dots - mx)
    ssum = jnp.sum(e, axis=1, keepdims=True)
    p = e / ssum
    so_ref[0, 0] = lax.dot_general(p, v, (((1,), (0,)), ((), ())),
                                   preferred_element_type=jnp.float32,
                                   precision=lax.Precision.DEFAULT)
    slog_ref[0, 0] = jnp.broadcast_to(jnp.log(ssum) + mx, (BUCKET_SIZE, 128))


def _attn_stage(sqk, sv, st):
    B = sqk.shape[0]
    grid = (B, N_CHUNKS)
    data_spec_c = pl.BlockSpec((1, 1, BUCKET_SIZE, D), lambda b, i: (b, i, 0, 0))
    data_spec_p = pl.BlockSpec((1, 1, BUCKET_SIZE, D),
                               lambda b, i: (b, (i - 1) % N_CHUNKS, 0, 0))
    st_spec_c = pl.BlockSpec((1, 1, 1, BUCKET_SIZE), lambda b, i: (b, i, 0, 0))
    st_spec_p = pl.BlockSpec((1, 1, 1, BUCKET_SIZE),
                             lambda b, i: (b, (i - 1) % N_CHUNKS, 0, 0))
    so, slog = pl.pallas_call(
        _attn_kernel,
        grid=grid,
        in_specs=[data_spec_c, data_spec_p, data_spec_c, data_spec_p,
                  st_spec_c, st_spec_p],
        out_specs=[
            pl.BlockSpec((1, 1, BUCKET_SIZE, D), lambda b, i: (b, i, 0, 0)),
            pl.BlockSpec((1, 1, BUCKET_SIZE, 128), lambda b, i: (b, i, 0, 0)),
        ],
        out_shape=[
            jax.ShapeDtypeStruct((B, N_CHUNKS, BUCKET_SIZE, D), jnp.float32),
            jax.ShapeDtypeStruct((B, N_CHUNKS, BUCKET_SIZE, 128), jnp.float32),
        ],
    )(sqk, sqk, sv, sv, st, st)
    return so, slog


def kernel(qk, v, seed_):
    B = qk.shape[0]
    rot = jax.random.normal(jax.random.key(seed_),
                            (B, D, N_HASHES, N_BUCKETS // 2), dtype=qk.dtype)
    rot2 = rot.reshape(B, D, N_HASHES * (N_BUCKETS // 2))

    buckets, keys = _hash_stage(qk, rot2)
    buckets_out = buckets.reshape(B, N_HASHES * S)

    st_local = jnp.argsort(keys, axis=-1).astype(jnp.int32)
    inv = jnp.argsort(st_local, axis=-1).astype(jnp.int32)

    st_flat = st_local.reshape(B, N_HASHES * S)
    gidx = (st_local
            + (jnp.arange(B, dtype=jnp.int32) * S)[:, None, None]).reshape(-1)
    sqk_f, sv_f = _sc_gather(qk.reshape(B * S, D), v.reshape(B * S, D), gidx)

    sqk = sqk_f.reshape(B, N_CHUNKS, BUCKET_SIZE, D)
    sv = sv_f.reshape(B, N_CHUNKS, BUCKET_SIZE, D)
    st4 = st_flat.reshape(B, N_CHUNKS, 1, BUCKET_SIZE)

    so, slog = _attn_stage(sqk, sv, st4)
    so_flat = so.reshape(B * N_HASHES * S, D)
    slogw = slog.reshape(B * N_HASHES * S, 128)

    idx_o = (inv
             + (jnp.arange(N_HASHES, dtype=jnp.int32) * S)[None, :, None]
             + (jnp.arange(B, dtype=jnp.int32) * (N_HASHES * S))[:, None, None])
    idx_o = jnp.transpose(idx_o, (0, 2, 1)).reshape(-1)
    out = _sc_combine(idx_o, so_flat, slogw).reshape(B, S, D)
    return out, buckets_out


# speedup vs baseline: 4778.8718x; 1.4243x over previous
import functools

import jax
import jax.numpy as jnp
from jax import lax
from jax.experimental import pallas as pl
from jax.experimental.pallas import tpu as pltpu
from jax.experimental.pallas import tpu_sc as plsc

N_HASHES = 8
BUCKET_SIZE = 64
N_BUCKETS = 64
S = 4096
D = 1024
HASH_TILE = 256
N_CHUNKS = N_HASHES * N_BUCKETS


def _hash_kernel(qk_ref, rot_ref, buckets_ref, keys_ref):
    s_blk = pl.program_id(1)
    x = qk_ref[0]
    r = rot_ref[0]
    rr = lax.dot_general(x, r, (((1,), (0,)), ((), ())),
                         preferred_element_type=jnp.float32,
                         precision=lax.Precision.DEFAULT)
    lane = lax.broadcasted_iota(jnp.int32, (HASH_TILE, N_BUCKETS // 2), 1)
    t_vec = s_blk * HASH_TILE + lax.broadcasted_iota(jnp.int32, (HASH_TILE,), 0)
    half = N_BUCKETS // 2
    for h in range(N_HASHES):
        sub = rr[:, h * half:(h + 1) * half]
        mx = jnp.max(sub, axis=1, keepdims=True)
        mn = jnp.min(sub, axis=1, keepdims=True)
        pos = jnp.min(jnp.where(sub == mx, lane, N_BUCKETS), axis=1)
        neg = jnp.min(jnp.where(sub == mn, lane, N_BUCKETS), axis=1)
        b_loc = jnp.where(mx[:, 0] >= -mn[:, 0], pos, half + neg)
        buckets_ref[0, h, :] = b_loc + h * N_BUCKETS
        keys_ref[0, h, :] = b_loc * S + t_vec


def _hash_stage(qk, rot):
    B = qk.shape[0]
    grid = (B, S // HASH_TILE)
    buckets, keys = pl.pallas_call(
        _hash_kernel,
        grid=grid,
        in_specs=[
            pl.BlockSpec((1, HASH_TILE, D), lambda b, s: (b, s, 0)),
            pl.BlockSpec((1, D, N_HASHES * (N_BUCKETS // 2)), lambda b, s: (b, 0, 0)),
        ],
        out_specs=[
            pl.BlockSpec((1, N_HASHES, HASH_TILE), lambda b, s: (b, 0, s)),
            pl.BlockSpec((1, N_HASHES, HASH_TILE), lambda b, s: (b, 0, s)),
        ],
        out_shape=[
            jax.ShapeDtypeStruct((B, N_HASHES, S), jnp.int32),
            jax.ShapeDtypeStruct((B, N_HASHES, S), jnp.int32),
        ],
    )(qk, rot)
    return buckets, keys


_N_WORKERS = 32
_GC = 32


def _sc_gather_body(qk_hbm, v_hbm, idx_hbm, sqk_hbm, sv_hbm,
                    idx_v, qkr, vr, sem_q, sem_v):
    n_rows = sqk_hbm.shape[0]
    rows_per_w = n_rows // _N_WORKERS
    wid = lax.axis_index("s") * 2 + lax.axis_index("c")
    base = wid * rows_per_w

    def step(c, carry):
        off = base + c * _GC
        pltpu.sync_copy(idx_hbm.at[pl.ds(off, _GC)], idx_v)
        cq = pltpu.async_copy(qk_hbm.at[idx_v], qkr, sem_q)
        cv = pltpu.async_copy(v_hbm.at[idx_v], vr, sem_v)
        cq.wait()
        cv.wait()
        pltpu.sync_copy(qkr, sqk_hbm.at[pl.ds(off, _GC)])
        pltpu.sync_copy(vr, sv_hbm.at[pl.ds(off, _GC)])
        return carry

    lax.fori_loop(0, rows_per_w // _GC, step, 0)


def _sc_gather(qk2, v2, idx):
    n = idx.shape[0]
    f = jax.ShapeDtypeStruct((n, D), jnp.float32)
    run = pl.kernel(
        _sc_gather_body,
        out_type=[f, f],
        mesh=plsc.VectorSubcoreMesh(core_axis_name="c", subcore_axis_name="s"),
        scratch_types=[
            pltpu.VMEM((_GC,), jnp.int32),
            pltpu.VMEM((_GC, D), jnp.float32),
            pltpu.VMEM((_GC, D), jnp.float32),
            pltpu.SemaphoreType.DMA,
            pltpu.SemaphoreType.DMA,
        ],
    )
    return run(qk2, v2, idx)


_TC8 = 8


def _sc_combine_body(idx_hbm, so_hbm, slogw_hbm, out_hbm,
                     idxbuf, rows, lrows, outbuf, sem, seml):
    total_t = out_hbm.shape[0]
    t_per_w = total_t // _N_WORKERS
    wid = lax.axis_index("s") * 2 + lax.axis_index("c")
    tbase = wid * t_per_w

    def chunk_step(c, carry):
        pltpu.sync_copy(idx_hbm.at[pl.ds((tbase + c * _TC8) * N_HASHES,
                                         _TC8 * N_HASHES)], idxbuf)
        cr = pltpu.async_copy(so_hbm.at[idxbuf], rows, sem)
        cl = pltpu.async_copy(slogw_hbm.at[idxbuf], lrows, seml)
        cr.wait()
        cl.wait()
        for tt in range(_TC8):
            ls = [lrows[tt * N_HASHES + h, pl.ds(0, 16)] for h in range(N_HASHES)]
            m = ls[0]
            for h in range(1, N_HASHES):
                m = jnp.maximum(m, ls[h])
            es = [jnp.exp(l - m) for l in ls]
            ssum = es[0]
            for h in range(1, N_HASHES):
                ssum = ssum + es[h]
            ws = [e / ssum for e in es]

            def d_step(dc, carry2):
                sl = pl.ds(dc * 16, 16)
                acc = ws[0] * rows[tt * N_HASHES, sl]
                for h in range(1, N_HASHES):
                    acc = acc + ws[h] * rows[tt * N_HASHES + h, sl]
                outbuf[tt, sl] = acc
                return carry2

            lax.fori_loop(0, D // 16, d_step, 0)
        pltpu.sync_copy(outbuf, out_hbm.at[pl.ds(tbase + c * _TC8, _TC8)])
        return carry

    lax.fori_loop(0, t_per_w // _TC8, chunk_step, 0)


def _sc_combine(idx_o, so_flat, slogw):
    total_t = so_flat.shape[0] // N_HASHES
    run = pl.kernel(
        _sc_combine_body,
        out_type=jax.ShapeDtypeStruct((total_t, D), jnp.float32),
        mesh=plsc.VectorSubcoreMesh(core_axis_name="c", subcore_axis_name="s"),
        scratch_types=[
            pltpu.VMEM((_TC8 * N_HASHES,), jnp.int32),
            pltpu.VMEM((_TC8 * N_HASHES, D), jnp.float32),
            pltpu.VMEM((_TC8 * N_HASHES, 128), jnp.float32),
            pltpu.VMEM((_TC8, D), jnp.float32),
            pltpu.SemaphoreType.DMA,
            pltpu.SemaphoreType.DMA,
        ],
    )
    return run(idx_o, so_flat, slogw)


_CB = 8


def _attn_kernel(sqk_c, sqk_p, sv_c, sv_p, st_c, st_p, so_ref, slog_ref):
    def unit(x):
        return x * jax.lax.rsqrt(jnp.sum(x * x, axis=1, keepdims=True) + 1e-6)

    kn = [unit(sqk_p[0, 0])] + [unit(sqk_c[0, j]) for j in range(_CB)]
    vs = [sv_p[0, 0]] + [sv_c[0, j] for j in range(_CB)]
    kt = [st_p[0, 0, 0]] + [st_c[0, j, 0] for j in range(_CB)]
    for j in range(_CB):
        q = sqk_c[0, j]
        k = jnp.concatenate([kn[j + 1], kn[j]], axis=0)
        v = jnp.concatenate([vs[j + 1], vs[j]], axis=0)
        dots = lax.dot_general(q, k, (((1,), (1,)), ((), ())),
                               preferred_element_type=jnp.float32,
                               precision=lax.Precision.DEFAULT) * (D ** -0.5)
        qt = kt[j + 1]
        ktj = jnp.concatenate([kt[j + 1], kt[j]])
        dots = jnp.where(qt[:, None] == ktj[None, :], -50000.0, dots)
        mx = jnp.max(dots, axis=1, keepdims=True)
        e = jnp.exp(dots - mx)
        ssum = jnp.sum(e, axis=1, keepdims=True)
        p = e / ssum
        so_ref[0, j] = lax.dot_general(p, v, (((1,), (0,)), ((), ())),
                                       preferred_element_type=jnp.float32,
                                       precision=lax.Precision.DEFAULT)
        slog_ref[0, j] = jnp.broadcast_to(jnp.log(ssum) + mx,
                                          (BUCKET_SIZE, 128))


def _attn_stage(sqk, sv, st):
    B = sqk.shape[0]
    grid = (B, N_CHUNKS // _CB)
    data_spec_c = pl.BlockSpec((1, _CB, BUCKET_SIZE, D),
                               lambda b, i: (b, i, 0, 0))
    data_spec_p = pl.BlockSpec((1, 1, BUCKET_SIZE, D),
                               lambda b, i: (b, (i * _CB - 1) % N_CHUNKS, 0, 0))
    st_spec_c = pl.BlockSpec((1, _CB, 1, BUCKET_SIZE), lambda b, i: (b, i, 0, 0))
    st_spec_p = pl.BlockSpec((1, 1, 1, BUCKET_SIZE),
                             lambda b, i: (b, (i * _CB - 1) % N_CHUNKS, 0, 0))
    so, slog = pl.pallas_call(
        _attn_kernel,
        grid=grid,
        in_specs=[data_spec_c, data_spec_p, data_spec_c, data_spec_p,
                  st_spec_c, st_spec_p],
        out_specs=[
            pl.BlockSpec((1, _CB, BUCKET_SIZE, D), lambda b, i: (b, i, 0, 0)),
            pl.BlockSpec((1, _CB, BUCKET_SIZE, 128), lambda b, i: (b, i, 0, 0)),
        ],
        out_shape=[
            jax.ShapeDtypeStruct((B, N_CHUNKS, BUCKET_SIZE, D), jnp.float32),
            jax.ShapeDtypeStruct((B, N_CHUNKS, BUCKET_SIZE, 128), jnp.float32),
        ],
    )(sqk, sqk, sv, sv, st, st)
    return so, slog


def kernel(qk, v, seed_):
    B = qk.shape[0]
    rot = jax.random.normal(jax.random.key(seed_),
                            (B, D, N_HASHES, N_BUCKETS // 2), dtype=qk.dtype)
    rot2 = rot.reshape(B, D, N_HASHES * (N_BUCKETS // 2))

    buckets, keys = _hash_stage(qk, rot2)
    buckets_out = buckets.reshape(B, N_HASHES * S)

    st_local = jnp.argsort(keys, axis=-1).astype(jnp.int32)
    inv = jnp.argsort(st_local, axis=-1).astype(jnp.int32)

    st_flat = st_local.reshape(B, N_HASHES * S)
    gidx = (st_local
            + (jnp.arange(B, dtype=jnp.int32) * S)[:, None, None]).reshape(-1)
    sqk_f, sv_f = _sc_gather(qk.reshape(B * S, D), v.reshape(B * S, D), gidx)

    sqk = sqk_f.reshape(B, N_CHUNKS, BUCKET_SIZE, D)
    sv = sv_f.reshape(B, N_CHUNKS, BUCKET_SIZE, D)
    st4 = st_flat.reshape(B, N_CHUNKS, 1, BUCKET_SIZE)

    so, slog = _attn_stage(sqk, sv, st4)
    so_flat = so.reshape(B * N_HASHES * S, D)
    slogw = slog.reshape(B * N_HASHES * S, 128)

    idx_o = (inv
             + (jnp.arange(N_HASHES, dtype=jnp.int32) * S)[None, :, None]
             + (jnp.arange(B, dtype=jnp.int32) * (N_HASHES * S))[:, None, None])
    idx_o = jnp.transpose(idx_o, (0, 2, 1)).reshape(-1)
    out = _sc_combine(idx_o, so_flat, slogw).reshape(B, S, D)
    return out, buckets_out


# speedup vs baseline: 4977.5020x; 1.0416x over previous
import functools

import jax
import jax.numpy as jnp
from jax import lax
from jax.experimental import pallas as pl
from jax.experimental.pallas import tpu as pltpu
from jax.experimental.pallas import tpu_sc as plsc

N_HASHES = 8
BUCKET_SIZE = 64
N_BUCKETS = 64
S = 4096
D = 1024
HASH_TILE = 256
N_CHUNKS = N_HASHES * N_BUCKETS


def _hash_kernel(qk_ref, rot_ref, buckets_ref, keys_ref):
    s_blk = pl.program_id(1)
    x = qk_ref[0]
    r = rot_ref[0]
    rr = lax.dot_general(x, r, (((1,), (0,)), ((), ())),
                         preferred_element_type=jnp.float32,
                         precision=lax.Precision.DEFAULT)
    lane = lax.broadcasted_iota(jnp.int32, (HASH_TILE, N_BUCKETS // 2), 1)
    t_vec = s_blk * HASH_TILE + lax.broadcasted_iota(jnp.int32, (HASH_TILE,), 0)
    half = N_BUCKETS // 2
    for h in range(N_HASHES):
        sub = rr[:, h * half:(h + 1) * half]
        mx = jnp.max(sub, axis=1, keepdims=True)
        mn = jnp.min(sub, axis=1, keepdims=True)
        pos = jnp.min(jnp.where(sub == mx, lane, N_BUCKETS), axis=1)
        neg = jnp.min(jnp.where(sub == mn, lane, N_BUCKETS), axis=1)
        b_loc = jnp.where(mx[:, 0] >= -mn[:, 0], pos, half + neg)
        buckets_ref[0, h, :] = b_loc + h * N_BUCKETS
        keys_ref[0, h, :] = b_loc * S + t_vec


def _hash_stage(qk, rot):
    B = qk.shape[0]
    grid = (B, S // HASH_TILE)
    buckets, keys = pl.pallas_call(
        _hash_kernel,
        grid=grid,
        in_specs=[
            pl.BlockSpec((1, HASH_TILE, D), lambda b, s: (b, s, 0)),
            pl.BlockSpec((1, D, N_HASHES * (N_BUCKETS // 2)), lambda b, s: (b, 0, 0)),
        ],
        out_specs=[
            pl.BlockSpec((1, N_HASHES, HASH_TILE), lambda b, s: (b, 0, s)),
            pl.BlockSpec((1, N_HASHES, HASH_TILE), lambda b, s: (b, 0, s)),
        ],
        out_shape=[
            jax.ShapeDtypeStruct((B, N_HASHES, S), jnp.int32),
            jax.ShapeDtypeStruct((B, N_HASHES, S), jnp.int32),
        ],
    )(qk, rot)
    return buckets, keys


_N_WORKERS = 32
_GC = 16


def _sc_gather_body(qk_hbm, v_hbm, idx_hbm, sqk_hbm, sv_hbm,
                    idx_all, qkr, vr,
                    gq0, gq1, gv0, gv1, sq0, sq1, sv0, sv1):
    gq = (gq0, gq1)
    gv = (gv0, gv1)
    sq = (sq0, sq1)
    svs = (sv0, sv1)
    rows_per_w = sqk_hbm.shape[0] // _N_WORKERS
    n_steps = rows_per_w // _GC
    wid = lax.axis_index("s") * 2 + lax.axis_index("c")
    base = wid * rows_per_w
    pltpu.sync_copy(idx_hbm.at[pl.ds(base, rows_per_w)], idx_all)

    def idxs(c):
        return idx_all.at[pl.ds(c * _GC, _GC)]

    def issue_gather(par, c):
        pltpu.async_copy(qk_hbm.at[idxs(c)], qkr.at[par], gq[par])
        pltpu.async_copy(v_hbm.at[idxs(c)], vr.at[par], gv[par])

    def wait_gather(par, c):
        pltpu.make_async_copy(qk_hbm.at[idxs(c)], qkr.at[par], gq[par]).wait()
        pltpu.make_async_copy(v_hbm.at[idxs(c)], vr.at[par], gv[par]).wait()

    def issue_store(par, c):
        off = base + c * _GC
        pltpu.async_copy(qkr.at[par], sqk_hbm.at[pl.ds(off, _GC)], sq[par])
        pltpu.async_copy(vr.at[par], sv_hbm.at[pl.ds(off, _GC)], svs[par])

    def wait_store(par, c):
        off = base + c * _GC
        pltpu.make_async_copy(qkr.at[par], sqk_hbm.at[pl.ds(off, _GC)],
                              sq[par]).wait()
        pltpu.make_async_copy(vr.at[par], sv_hbm.at[pl.ds(off, _GC)],
                              svs[par]).wait()

    issue_gather(0, 0)

    def step2(c2, carry):
        for par in (0, 1):
            c = c2 * 2 + par
            other = 1 - par

            wait_gather(par, c)

            @pl.when(c + 1 < n_steps)
            def _():
                @pl.when(c >= 1)
                def _():
                    wait_store(other, c - 1)
                issue_gather(other, c + 1)

            issue_store(par, c)
        return carry

    lax.fori_loop(0, n_steps // 2, step2, 0)
    wait_store(0, n_steps - 2)
    wait_store(1, n_steps - 1)


def _sc_gather(qk2, v2, idx):
    n = idx.shape[0]
    f = jax.ShapeDtypeStruct((n, D), jnp.float32)
    run = pl.kernel(
        _sc_gather_body,
        out_type=[f, f],
        mesh=plsc.VectorSubcoreMesh(core_axis_name="c", subcore_axis_name="s"),
        scratch_types=[
            pltpu.VMEM((n // _N_WORKERS,), jnp.int32),
            pltpu.VMEM((2, _GC, D), jnp.float32),
            pltpu.VMEM((2, _GC, D), jnp.float32),
            pltpu.SemaphoreType.DMA,
            pltpu.SemaphoreType.DMA,
            pltpu.SemaphoreType.DMA,
            pltpu.SemaphoreType.DMA,
            pltpu.SemaphoreType.DMA,
            pltpu.SemaphoreType.DMA,
            pltpu.SemaphoreType.DMA,
            pltpu.SemaphoreType.DMA,
        ],
    )
    return run(qk2, v2, idx)


_TC8 = 8


def _sc_combine_body(idx_hbm, so_hbm, slogw_hbm, out_hbm,
                     idxbuf, rows, lrows, outbuf, sem, seml):
    total_t = out_hbm.shape[0]
    t_per_w = total_t // _N_WORKERS
    wid = lax.axis_index("s") * 2 + lax.axis_index("c")
    tbase = wid * t_per_w

    def chunk_step(c, carry):
        pltpu.sync_copy(idx_hbm.at[pl.ds((tbase + c * _TC8) * N_HASHES,
                                         _TC8 * N_HASHES)], idxbuf)
        cr = pltpu.async_copy(so_hbm.at[idxbuf], rows, sem)
        cl = pltpu.async_copy(slogw_hbm.at[idxbuf], lrows, seml)
        cr.wait()
        cl.wait()
        for tt in range(_TC8):
            ls = [lrows[tt * N_HASHES + h, pl.ds(0, 16)] for h in range(N_HASHES)]
            m = ls[0]
            for h in range(1, N_HASHES):
                m = jnp.maximum(m, ls[h])
            es = [jnp.exp(l - m) for l in ls]
            ssum = es[0]
            for h in range(1, N_HASHES):
                ssum = ssum + es[h]
            ws = [e / ssum for e in es]

            def d_step(dc, carry2):
                sl = pl.ds(dc * 16, 16)
                acc = ws[0] * rows[tt * N_HASHES, sl]
                for h in range(1, N_HASHES):
                    acc = acc + ws[h] * rows[tt * N_HASHES + h, sl]
                outbuf[tt, sl] = acc
                return carry2

            lax.fori_loop(0, D // 16, d_step, 0)
        pltpu.sync_copy(outbuf, out_hbm.at[pl.ds(tbase + c * _TC8, _TC8)])
        return carry

    lax.fori_loop(0, t_per_w // _TC8, chunk_step, 0)


def _sc_combine(idx_o, so_flat, slogw):
    total_t = so_flat.shape[0] // N_HASHES
    run = pl.kernel(
        _sc_combine_body,
        out_type=jax.ShapeDtypeStruct((total_t, D), jnp.float32),
        mesh=plsc.VectorSubcoreMesh(core_axis_name="c", subcore_axis_name="s"),
        scratch_types=[
            pltpu.VMEM((_TC8 * N_HASHES,), jnp.int32),
            pltpu.VMEM((_TC8 * N_HASHES, D), jnp.float32),
            pltpu.VMEM((_TC8 * N_HASHES, 128), jnp.float32),
            pltpu.VMEM((_TC8, D), jnp.float32),
            pltpu.SemaphoreType.DMA,
            pltpu.SemaphoreType.DMA,
        ],
    )
    return run(idx_o, so_flat, slogw)


_CB = 16


def _attn_kernel(sqk_c, sqk_p, sv_c, sv_p, st_c, st_p, so_ref, slog_ref):
    def unit(x):
        return x * jax.lax.rsqrt(jnp.sum(x * x, axis=1, keepdims=True) + 1e-6)

    kn = [unit(sqk_p[0, 0])] + [unit(sqk_c[0, j]) for j in range(_CB)]
    vs = [sv_p[0, 0]] + [sv_c[0, j] for j in range(_CB)]
    kt = [st_p[0, 0, 0]] + [st_c[0, j, 0] for j in range(_CB)]
    for j in range(_CB):
        q = sqk_c[0, j]
        k = jnp.concatenate([kn[j + 1], kn[j]], axis=0)
        v = jnp.concatenate([vs[j + 1], vs[j]], axis=0)
        dots = lax.dot_general(q, k, (((1,), (1,)), ((), ())),
                               preferred_element_type=jnp.float32,
                               precision=lax.Precision.DEFAULT) * (D ** -0.5)
        qt = kt[j + 1]
        ktj = jnp.concatenate([kt[j + 1], kt[j]])
        dots = jnp.where(qt[:, None] == ktj[None, :], -50000.0, dots)
        mx = jnp.max(dots, axis=1, keepdims=True)
        e = jnp.exp(dots - mx)
        ssum = jnp.sum(e, axis=1, keepdims=True)
        p = e / ssum
        so_ref[0, j] = lax.dot_general(p, v, (((1,), (0,)), ((), ())),
                                       preferred_element_type=jnp.float32,
                                       precision=lax.Precision.DEFAULT)
        slog_ref[0, j] = jnp.broadcast_to(jnp.log(ssum) + mx,
                                          (BUCKET_SIZE, 128))


def _attn_stage(sqk, sv, st):
    B = sqk.shape[0]
    grid = (B, N_CHUNKS // _CB)
    data_spec_c = pl.BlockSpec((1, _CB, BUCKET_SIZE, D),
                               lambda b, i: (b, i, 0, 0))
    data_spec_p = pl.BlockSpec((1, 1, BUCKET_SIZE, D),
                               lambda b, i: (b, (i * _CB - 1) % N_CHUNKS, 0, 0))
    st_spec_c = pl.BlockSpec((1, _CB, 1, BUCKET_SIZE), lambda b, i: (b, i, 0, 0))
    st_spec_p = pl.BlockSpec((1, 1, 1, BUCKET_SIZE),
                             lambda b, i: (b, (i * _CB - 1) % N_CHUNKS, 0, 0))
    so, slog = pl.pallas_call(
        _attn_kernel,
        grid=grid,
        in_specs=[data_spec_c, data_spec_p, data_spec_c, data_spec_p,
                  st_spec_c, st_spec_p],
        out_specs=[
            pl.BlockSpec((1, _CB, BUCKET_SIZE, D), lambda b, i: (b, i, 0, 0)),
            pl.BlockSpec((1, _CB, BUCKET_SIZE, 128), lambda b, i: (b, i, 0, 0)),
        ],
        out_shape=[
            jax.ShapeDtypeStruct((B, N_CHUNKS, BUCKET_SIZE, D), jnp.float32),
            jax.ShapeDtypeStruct((B, N_CHUNKS, BUCKET_SIZE, 128), jnp.float32),
        ],
    )(sqk, sqk, sv, sv, st, st)
    return so, slog


def kernel(qk, v, seed_):
    B = qk.shape[0]
    rot = jax.random.normal(jax.random.key(seed_),
                            (B, D, N_HASHES, N_BUCKETS // 2), dtype=qk.dtype)
    rot2 = rot.reshape(B, D, N_HASHES * (N_BUCKETS // 2))

    buckets, keys = _hash_stage(qk, rot2)
    buckets_out = buckets.reshape(B, N_HASHES * S)

    st_local = jnp.argsort(keys, axis=-1).astype(jnp.int32)
    inv = jnp.argsort(st_local, axis=-1).astype(jnp.int32)

    st_flat = st_local.reshape(B, N_HASHES * S)
    gidx = (st_local
            + (jnp.arange(B, dtype=jnp.int32) * S)[:, None, None]).reshape(-1)
    sqk_f, sv_f = _sc_gather(qk.reshape(B * S, D), v.reshape(B * S, D), gidx)

    sqk = sqk_f.reshape(B, N_CHUNKS, BUCKET_SIZE, D)
    sv = sv_f.reshape(B, N_CHUNKS, BUCKET_SIZE, D)
    st4 = st_flat.reshape(B, N_CHUNKS, 1, BUCKET_SIZE)

    so, slog = _attn_stage(sqk, sv, st4)
    so_flat = so.reshape(B * N_HASHES * S, D)
    slogw = slog.reshape(B * N_HASHES * S, 128)

    idx_o = (inv
             + (jnp.arange(N_HASHES, dtype=jnp.int32) * S)[None, :, None]
             + (jnp.arange(B, dtype=jnp.int32) * (N_HASHES * S))[:, None, None])
    idx_o = jnp.transpose(idx_o, (0, 2, 1)).reshape(-1)
    out = _sc_combine(idx_o, so_flat, slogw).reshape(B, S, D)
    return out, buckets_out


# speedup vs baseline: 5076.4039x; 1.0199x over previous
import functools

import jax
import jax.numpy as jnp
from jax import lax
from jax.experimental import pallas as pl
from jax.experimental.pallas import tpu as pltpu
from jax.experimental.pallas import tpu_sc as plsc

N_HASHES = 8
BUCKET_SIZE = 64
N_BUCKETS = 64
S = 4096
D = 1024
HASH_TILE = 512
N_CHUNKS = N_HASHES * N_BUCKETS


def _hash_kernel(qk_ref, rot_ref, buckets_ref, keys_ref):
    s_blk = pl.program_id(1)
    x = qk_ref[0]
    r = rot_ref[0]
    rr = lax.dot_general(x, r, (((1,), (0,)), ((), ())),
                         preferred_element_type=jnp.float32,
                         precision=lax.Precision.DEFAULT)
    lane = lax.broadcasted_iota(jnp.int32, (HASH_TILE, N_BUCKETS // 2), 1)
    t_vec = s_blk * HASH_TILE + lax.broadcasted_iota(jnp.int32, (HASH_TILE,), 0)
    half = N_BUCKETS // 2
    for h in range(N_HASHES):
        sub = rr[:, h * half:(h + 1) * half]
        mx = jnp.max(sub, axis=1, keepdims=True)
        mn = jnp.min(sub, axis=1, keepdims=True)
        pos = jnp.min(jnp.where(sub == mx, lane, N_BUCKETS), axis=1)
        neg = jnp.min(jnp.where(sub == mn, lane, N_BUCKETS), axis=1)
        b_loc = jnp.where(mx[:, 0] >= -mn[:, 0], pos, half + neg)
        buckets_ref[0, h, :] = b_loc + h * N_BUCKETS
        keys_ref[0, h, :] = b_loc * S + t_vec


def _hash_stage(qk, rot):
    B = qk.shape[0]
    grid = (B, S // HASH_TILE)
    buckets, keys = pl.pallas_call(
        _hash_kernel,
        grid=grid,
        in_specs=[
            pl.BlockSpec((1, HASH_TILE, D), lambda b, s: (b, s, 0)),
            pl.BlockSpec((1, D, N_HASHES * (N_BUCKETS // 2)), lambda b, s: (b, 0, 0)),
        ],
        out_specs=[
            pl.BlockSpec((1, N_HASHES, HASH_TILE), lambda b, s: (b, 0, s)),
            pl.BlockSpec((1, N_HASHES, HASH_TILE), lambda b, s: (b, 0, s)),
        ],
        out_shape=[
            jax.ShapeDtypeStruct((B, N_HASHES, S), jnp.int32),
            jax.ShapeDtypeStruct((B, N_HASHES, S), jnp.int32),
        ],
    )(qk, rot)
    return buckets, keys


_N_WORKERS = 32
_GC = 16


def _sc_gather_body(qk_hbm, v_hbm, idx_hbm, sqk_hbm, sv_hbm,
                    idx_all, qkr, vr,
                    gq0, gq1, gv0, gv1, sq0, sq1, sv0, sv1):
    gq = (gq0, gq1)
    gv = (gv0, gv1)
    sq = (sq0, sq1)
    svs = (sv0, sv1)
    rows_per_w = sqk_hbm.shape[0] // _N_WORKERS
    n_steps = rows_per_w // _GC
    wid = lax.axis_index("s") * 2 + lax.axis_index("c")
    base = wid * rows_per_w
    pltpu.sync_copy(idx_hbm.at[pl.ds(base, rows_per_w)], idx_all)

    def idxs(c):
        return idx_all.at[pl.ds(c * _GC, _GC)]

    def issue_gather(par, c):
        pltpu.async_copy(qk_hbm.at[idxs(c)], qkr.at[par], gq[par])
        pltpu.async_copy(v_hbm.at[idxs(c)], vr.at[par], gv[par])

    def wait_gather(par, c):
        pltpu.make_async_copy(qk_hbm.at[idxs(c)], qkr.at[par], gq[par]).wait()
        pltpu.make_async_copy(v_hbm.at[idxs(c)], vr.at[par], gv[par]).wait()

    def issue_store(par, c):
        off = base + c * _GC
        pltpu.async_copy(qkr.at[par], sqk_hbm.at[pl.ds(off, _GC)], sq[par])
        pltpu.async_copy(vr.at[par], sv_hbm.at[pl.ds(off, _GC)], svs[par])

    def wait_store(par, c):
        off = base + c * _GC
        pltpu.make_async_copy(qkr.at[par], sqk_hbm.at[pl.ds(off, _GC)],
                              sq[par]).wait()
        pltpu.make_async_copy(vr.at[par], sv_hbm.at[pl.ds(off, _GC)],
                              svs[par]).wait()

    issue_gather(0, 0)

    def step2(c2, carry):
        for par in (0, 1):
            c = c2 * 2 + par
            other = 1 - par

            wait_gather(par, c)

            @pl.when(c + 1 < n_steps)
            def _():
                @pl.when(c >= 1)
                def _():
                    wait_store(other, c - 1)
                issue_gather(other, c + 1)

            issue_store(par, c)
        return carry

    lax.fori_loop(0, n_steps // 2, step2, 0)
    wait_store(0, n_steps - 2)
    wait_store(1, n_steps - 1)


def _sc_gather(qk2, v2, idx):
    n = idx.shape[0]
    f = jax.ShapeDtypeStruct((n, D), jnp.float32)
    run = pl.kernel(
        _sc_gather_body,
        out_type=[f, f],
        mesh=plsc.VectorSubcoreMesh(core_axis_name="c", subcore_axis_name="s"),
        scratch_types=[
            pltpu.VMEM((n // _N_WORKERS,), jnp.int32),
            pltpu.VMEM((2, _GC, D), jnp.float32),
            pltpu.VMEM((2, _GC, D), jnp.float32),
            pltpu.SemaphoreType.DMA,
            pltpu.SemaphoreType.DMA,
            pltpu.SemaphoreType.DMA,
            pltpu.SemaphoreType.DMA,
            pltpu.SemaphoreType.DMA,
            pltpu.SemaphoreType.DMA,
            pltpu.SemaphoreType.DMA,
            pltpu.SemaphoreType.DMA,
        ],
    )
    return run(qk2, v2, idx)


_TC8 = 8


def _sc_combine_body(idx_hbm, so_hbm, slogw_hbm, out_hbm,
                     idxbuf, rows, lrows, outbuf, sem, seml):
    total_t = out_hbm.shape[0]
    t_per_w = total_t // _N_WORKERS
    wid = lax.axis_index("s") * 2 + lax.axis_index("c")
    tbase = wid * t_per_w

    def chunk_step(c, carry):
        pltpu.sync_copy(idx_hbm.at[pl.ds((tbase + c * _TC8) * N_HASHES,
                                         _TC8 * N_HASHES)], idxbuf)
        cr = pltpu.async_copy(so_hbm.at[idxbuf], rows, sem)
        cl = pltpu.async_copy(slogw_hbm.at[idxbuf], lrows, seml)
        cr.wait()
        cl.wait()
        for tt in range(_TC8):
            ls = [lrows[tt * N_HASHES + h, pl.ds(0, 16)] for h in range(N_HASHES)]
            m = ls[0]
            for h in range(1, N_HASHES):
                m = jnp.maximum(m, ls[h])
            es = [jnp.exp(l - m) for l in ls]
            ssum = es[0]
            for h in range(1, N_HASHES):
                ssum = ssum + es[h]
            ws = [e / ssum for e in es]

            def d_step(dc, carry2):
                sl = pl.ds(dc * 16, 16)
                acc = ws[0] * rows[tt * N_HASHES, sl]
                for h in range(1, N_HASHES):
                    acc = acc + ws[h] * rows[tt * N_HASHES + h, sl]
                outbuf[tt, sl] = acc
                return carry2

            lax.fori_loop(0, D // 16, d_step, 0)
        pltpu.sync_copy(outbuf, out_hbm.at[pl.ds(tbase + c * _TC8, _TC8)])
        return carry

    lax.fori_loop(0, t_per_w // _TC8, chunk_step, 0)


def _sc_combine(idx_o, so_flat, slogw):
    total_t = so_flat.shape[0] // N_HASHES
    run = pl.kernel(
        _sc_combine_body,
        out_type=jax.ShapeDtypeStruct((total_t, D), jnp.float32),
        mesh=plsc.VectorSubcoreMesh(core_axis_name="c", subcore_axis_name="s"),
        scratch_types=[
            pltpu.VMEM((_TC8 * N_HASHES,), jnp.int32),
            pltpu.VMEM((_TC8 * N_HASHES, D), jnp.float32),
            pltpu.VMEM((_TC8 * N_HASHES, 128), jnp.float32),
            pltpu.VMEM((_TC8, D), jnp.float32),
            pltpu.SemaphoreType.DMA,
            pltpu.SemaphoreType.DMA,
        ],
    )
    return run(idx_o, so_flat, slogw)


_CB = 16


def _attn_kernel(sqk_c, sqk_p, sv_c, sv_p, st_c, st_p, so_ref, slog_ref):
    def unit(x):
        return x * jax.lax.rsqrt(jnp.sum(x * x, axis=1, keepdims=True) + 1e-6)

    kn = [unit(sqk_p[0, 0])] + [unit(sqk_c[0, j]) for j in range(_CB)]
    vs = [sv_p[0, 0]] + [sv_c[0, j] for j in range(_CB)]
    kt = [st_p[0, 0, 0]] + [st_c[0, j, 0] for j in range(_CB)]
    for j in range(_CB):
        q = sqk_c[0, j]
        k = jnp.concatenate([kn[j + 1], kn[j]], axis=0)
        v = jnp.concatenate([vs[j + 1], vs[j]], axis=0)
        dots = lax.dot_general(q, k, (((1,), (1,)), ((), ())),
                               preferred_element_type=jnp.float32,
                               precision=lax.Precision.DEFAULT) * (D ** -0.5)
        qt = kt[j + 1]
        ktj = jnp.concatenate([kt[j + 1], kt[j]])
        dots = jnp.where(qt[:, None] == ktj[None, :], -50000.0, dots)
        mx = jnp.max(dots, axis=1, keepdims=True)
        e = jnp.exp(dots - mx)
        ssum = jnp.sum(e, axis=1, keepdims=True)
        p = e / ssum
        so_ref[0, j] = lax.dot_general(p, v, (((1,), (0,)), ((), ())),
                                       preferred_element_type=jnp.float32,
                                       precision=lax.Precision.DEFAULT)
        slog_ref[0, j] = jnp.broadcast_to(jnp.log(ssum) + mx,
                                          (BUCKET_SIZE, 128))


def _attn_stage(sqk, sv, st):
    B = sqk.shape[0]
    grid = (B, N_CHUNKS // _CB)
    data_spec_c = pl.BlockSpec((1, _CB, BUCKET_SIZE, D),
                               lambda b, i: (b, i, 0, 0))
    data_spec_p = pl.BlockSpec((1, 1, BUCKET_SIZE, D),
                               lambda b, i: (b, (i * _CB - 1) % N_CHUNKS, 0, 0))
    st_spec_c = pl.BlockSpec((1, _CB, 1, BUCKET_SIZE), lambda b, i: (b, i, 0, 0))
    st_spec_p = pl.BlockSpec((1, 1, 1, BUCKET_SIZE),
                             lambda b, i: (b, (i * _CB - 1) % N_CHUNKS, 0, 0))
    so, slog = pl.pallas_call(
        _attn_kernel,
        grid=grid,
        in_specs=[data_spec_c, data_spec_p, data_spec_c, data_spec_p,
                  st_spec_c, st_spec_p],
        out_specs=[
            pl.BlockSpec((1, _CB, BUCKET_SIZE, D), lambda b, i: (b, i, 0, 0)),
            pl.BlockSpec((1, _CB, BUCKET_SIZE, 128), lambda b, i: (b, i, 0, 0)),
        ],
        out_shape=[
            jax.ShapeDtypeStruct((B, N_CHUNKS, BUCKET_SIZE, D), jnp.float32),
            jax.ShapeDtypeStruct((B, N_CHUNKS, BUCKET_SIZE, 128), jnp.float32),
        ],
    )(sqk, sqk, sv, sv, st, st)
    return so, slog


def kernel(qk, v, seed_):
    B = qk.shape[0]
    rot = jax.random.normal(jax.random.key(seed_),
                            (B, D, N_HASHES, N_BUCKETS // 2), dtype=qk.dtype)
    rot2 = rot.reshape(B, D, N_HASHES * (N_BUCKETS // 2))

    buckets, keys = _hash_stage(qk, rot2)
    buckets_out = buckets.reshape(B, N_HASHES * S)

    st_local = jnp.argsort(keys, axis=-1).astype(jnp.int32)
    bidx = jnp.arange(B, dtype=jnp.int32)[:, None, None]
    hidx = jnp.arange(N_HASHES, dtype=jnp.int32)[None, :, None]
    iota_s = jnp.broadcast_to(jnp.arange(S, dtype=jnp.int32), (B, N_HASHES, S))
    inv = jnp.zeros((B, N_HASHES, S), jnp.int32).at[bidx, hidx, st_local].set(iota_s)

    st_flat = st_local.reshape(B, N_HASHES * S)
    gidx = (st_local
            + (jnp.arange(B, dtype=jnp.int32) * S)[:, None, None]).reshape(-1)
    sqk_f, sv_f = _sc_gather(qk.reshape(B * S, D), v.reshape(B * S, D), gidx)

    sqk = sqk_f.reshape(B, N_CHUNKS, BUCKET_SIZE, D)
    sv = sv_f.reshape(B, N_CHUNKS, BUCKET_SIZE, D)
    st4 = st_flat.reshape(B, N_CHUNKS, 1, BUCKET_SIZE)

    so, slog = _attn_stage(sqk, sv, st4)
    so_flat = so.reshape(B * N_HASHES * S, D)
    slogw = slog.reshape(B * N_HASHES * S, 128)

    idx_o = (inv
             + (jnp.arange(N_HASHES, dtype=jnp.int32) * S)[None, :, None]
             + (jnp.arange(B, dtype=jnp.int32) * (N_HASHES * S))[:, None, None])
    idx_o = jnp.transpose(idx_o, (0, 2, 1)).reshape(-1)
    out = _sc_combine(idx_o, so_flat, slogw).reshape(B, S, D)
    return out, buckets_out
